# Initial kernel scaffold; baseline (speedup 1.0000x reference)
import jax
import jax.numpy as jnp
from jax.experimental import pallas as pl


def kernel(x, edge_index, W1, b1, W2, b2, W3, b3):
    raise NotImplementedError("write your pallas kernel here")


# speedup vs baseline: 2.9047x; 2.9047x over previous
import functools

import jax
import jax.numpy as jnp
from jax import lax
from jax.experimental import pallas as pl
from jax.experimental.pallas import tpu as pltpu
from jax.experimental.pallas import tpu_sc as plsc

_N = 10000
_E = 320000
_CH = 128
_HC = 64
_L = 128
_NW = 32
_EPAD = 327680
_EBLK = _EPAD // _L
_J = _EBLK // _NW
_NACC = 10240
_DUMMY_DST = 10016
_RPT = _NACC // 16


def _sc_aggregate(table, src2d, dst2d, zblk):
    mesh = plsc.VectorSubcoreMesh(core_axis_name="c", subcore_axis_name="s")

    @functools.partial(
        pl.kernel,
        out_type=jax.ShapeDtypeStruct((2, _NACC, _HC), jnp.float32),
        mesh=mesh,
        compiler_params=pltpu.CompilerParams(use_tc_tiling_on_sc=False),
        scratch_types=[
            pltpu.VMEM((_J, _L), jnp.int32),
            pltpu.VMEM((_J, _L), jnp.int32),
            pltpu.VMEM((_L, _HC), jnp.float32),
            pltpu.VMEM((_L, _HC), jnp.float32),
            pltpu.VMEM_SHARED((_NACC, _HC), jnp.float32),
            pltpu.SemaphoreType.DMA,
        ],
    )
    def agg(table_h, src_h, dst_h, z_h, out_h, src_v, dst_v, rows_v, buf_v,
            acc_s, sem):
        c = lax.axis_index("c")
        s = lax.axis_index("s")
        wid = c * 16 + s

        pltpu.sync_copy(z_h, buf_v)
        for k in range(_RPT // _L):
            pltpu.sync_copy(buf_v, acc_s.at[pl.ds(s * _RPT + k * _L, _L)])

        pltpu.sync_copy(src_h.at[pl.ds(wid * _J, _J)], src_v)
        pltpu.sync_copy(dst_h.at[pl.ds(wid * _J, _J)], dst_v)

        plsc.subcore_barrier()

        def step(j, carry):
            pltpu.async_copy(table_h.at[src_v.at[j]], rows_v, sem).wait()
            pltpu.sync_copy(rows_v, acc_s.at[dst_v.at[j]], add=True)
            return carry

        lax.fori_loop(0, _J, step, 0)

        plsc.subcore_barrier()

        for k in range(_RPT // _L):
            r0 = s * _RPT + k * _L
            pltpu.sync_copy(acc_s.at[pl.ds(r0, _L)], buf_v)
            pltpu.sync_copy(buf_v, out_h.at[c, pl.ds(r0, _L)])

    return agg(table, src2d, dst2d, zblk)


_BLK = 1000


def _tc_layer1(agg_lo, agg_hi, W1lo, W1hi, b1):
    def body(lo_ref, hi_ref, wlo_ref, whi_ref, b_ref, hlo_ref, hhi_ref):
        a_lo = lo_ref[0] + lo_ref[1]
        a_hi = hi_ref[0] + hi_ref[1]
        h = jnp.maximum(
            jnp.dot(a_lo, wlo_ref[...], preferred_element_type=jnp.float32)
            + jnp.dot(a_hi, whi_ref[...], preferred_element_type=jnp.float32)
            + b_ref[...], 0.0)
        hlo_ref[...] = h[:, :_HC]
        hhi_ref[...] = h[:, _HC:]

    return pl.pallas_call(
        body,
        grid=(_N // _BLK,),
        in_specs=[
            pl.BlockSpec((2, _BLK, _HC), lambda i: (0, i, 0)),
            pl.BlockSpec((2, _BLK, _HC), lambda i: (0, i, 0)),
            pl.BlockSpec((_HC, _CH), lambda i: (0, 0)),
            pl.BlockSpec((_HC, _CH), lambda i: (0, 0)),
            pl.BlockSpec((1, _CH), lambda i: (0, 0)),
        ],
        out_specs=[
            pl.BlockSpec((_BLK, _HC), lambda i: (i, 0)),
            pl.BlockSpec((_BLK, _HC), lambda i: (i, 0)),
        ],
        out_shape=[
            jax.ShapeDtypeStruct((_N, _HC), jnp.float32),
            jax.ShapeDtypeStruct((_N, _HC), jnp.float32),
        ],
    )(agg_lo, agg_hi, W1lo, W1hi, b1.reshape(1, _CH))


def _tc_layer23(agg_lo, agg_hi, W2lo, W2hi, b2, W3lo, W3hi, b3):
    oc = W2lo.shape[1]

    def body(lo_ref, hi_ref, w2l, w2h, b2_ref, w3l, w3h, b3_ref,
             mu_ref, lv_ref):
        a_lo = lo_ref[0] + lo_ref[1]
        a_hi = hi_ref[0] + hi_ref[1]
        mu_ref[...] = (
            jnp.dot(a_lo, w2l[...], preferred_element_type=jnp.float32)
            + jnp.dot(a_hi, w2h[...], preferred_element_type=jnp.float32)
            + b2_ref[...])
        lv_ref[...] = (
            jnp.dot(a_lo, w3l[...], preferred_element_type=jnp.float32)
            + jnp.dot(a_hi, w3h[...], preferred_element_type=jnp.float32)
            + b3_ref[...])

    return pl.pallas_call(
        body,
        grid=(_N // _BLK,),
        in_specs=[
            pl.BlockSpec((2, _BLK, _HC), lambda i: (0, i, 0)),
            pl.BlockSpec((2, _BLK, _HC), lambda i: (0, i, 0)),
            pl.BlockSpec((_HC, oc), lambda i: (0, 0)),
            pl.BlockSpec((_HC, oc), lambda i: (0, 0)),
            pl.BlockSpec((1, oc), lambda i: (0, 0)),
            pl.BlockSpec((_HC, oc), lambda i: (0, 0)),
            pl.BlockSpec((_HC, oc), lambda i: (0, 0)),
            pl.BlockSpec((1, oc), lambda i: (0, 0)),
        ],
        out_specs=[
            pl.BlockSpec((_BLK, oc), lambda i: (i, 0)),
            pl.BlockSpec((_BLK, oc), lambda i: (i, 0)),
        ],
        out_shape=[
            jax.ShapeDtypeStruct((_N, oc), jnp.float32),
            jax.ShapeDtypeStruct((_N, oc), jnp.float32),
        ],
    )(agg_lo, agg_hi, W2lo, W2hi, b2.reshape(1, oc),
      W3lo, W3hi, b3.reshape(1, oc))


def kernel(x, edge_index, W1, b1, W2, b2, W3, b3):
    src = edge_index[0]
    dst = edge_index[1]
    pad = _EPAD - _E
    src2d = jnp.concatenate(
        [src, jnp.zeros((pad,), jnp.int32)]).reshape(_EBLK, _L)
    dst2d = jnp.concatenate(
        [dst, jnp.full((pad,), _DUMMY_DST, jnp.int32)]).reshape(_EBLK, _L)
    zblk = jnp.zeros((_L, _HC), jnp.float32)

    a1_lo = _sc_aggregate(x[:, :_HC], src2d, dst2d, zblk)
    a1_hi = _sc_aggregate(x[:, _HC:], src2d, dst2d, zblk)
    h_lo, h_hi = _tc_layer1(a1_lo, a1_hi, W1[:_HC, :], W1[_HC:, :], b1)
    a2_lo = _sc_aggregate(h_lo, src2d, dst2d, zblk)
    a2_hi = _sc_aggregate(h_hi, src2d, dst2d, zblk)
    mu, logvar = _tc_layer23(a2_lo, a2_hi, W2[:_HC, :], W2[_HC:, :], b2,
                             W3[:_HC, :], W3[_HC:, :], b3)
    return (mu, logvar)
